# Initial kernel scaffold; baseline (speedup 1.0000x reference)
import jax
import jax.numpy as jnp
from jax.experimental import pallas as pl


def kernel(code_vec, code_mask, doc_vec, doc_mask, W_code, W_doc):
    raise NotImplementedError("write your pallas kernel here")


# speedup vs baseline: 12.4262x; 12.4262x over previous
import functools

import jax
import jax.numpy as jnp
from jax import lax
from jax.experimental import pallas as pl
from jax.experimental.pallas import tpu as pltpu
from jax.experimental.pallas import tpu_sc as plsc

B, L, D, V = 4096, 200, 128, 32767
NC, NS, LANES = 2, 16, 16
NW = NC * NS
SPW = B // NW
HALF = 100
HPAD = 104
MPAD = 208
NV = D // LANES


def _splat(i):
    return jnp.full((LANES,), i, jnp.int32)


def _compute_denoms(mask_v, denom_v):
    def group_body(g, _):
        rows = (g * LANES + lax.iota(jnp.int32, LANES)) * MPAD

        def col_body(c, acc):
            return acc + plsc.load_gather(mask_v, [rows + c])

        tot = lax.fori_loop(0, MPAD, col_body, jnp.zeros((LANES,), jnp.float32))
        denom_v[pl.ds(g * LANES, LANES)] = 1.0 / jnp.maximum(tot, 1e-9)
        return 0

    lax.fori_loop(0, SPW // LANES, group_body, 0)


def _accumulate(rows_v, buf, i, mask_v, denom_v, out_v):
    si = _splat(i)
    mbase = _splat(i * MPAD)

    def row_body(l, accs):
        new = list(accs)
        for u in range(2):
            r = 2 * l + u
            m = plsc.load_gather(mask_v, [mbase + r])
            new = [
                new[j] + m * rows_v[buf, r, pl.ds(j * LANES, LANES)]
                for j in range(NV)
            ]
        return tuple(new)

    accs = lax.fori_loop(
        0, L // 2, row_body, tuple(jnp.zeros((LANES,), jnp.float32) for _ in range(NV))
    )

    r = plsc.load_gather(denom_v, [si])
    for j in range(NV):
        out_v[i, pl.ds(j * LANES, LANES)] = accs[j] * r


def _gather_pair(w_hbm, idx_v, rows_v, i, buf, sem):
    return (
        pltpu.make_async_copy(
            w_hbm.at[idx_v.at[i, 0, pl.ds(0, HALF)]],
            rows_v.at[buf, pl.ds(0, HALF)],
            sem,
        ),
        pltpu.make_async_copy(
            w_hbm.at[idx_v.at[i, 1, pl.ds(0, HALF)]],
            rows_v.at[buf, pl.ds(HALF, HALF)],
            sem,
        ),
    )


def _make_sc_kernel():
    mesh = plsc.VectorSubcoreMesh(core_axis_name="c", subcore_axis_name="s")
    f32 = jnp.float32

    @functools.partial(
        pl.kernel,
        mesh=mesh,
        compiler_params=pltpu.CompilerParams(needs_layout_passes=False),
        out_type=(
            jax.ShapeDtypeStruct((B, D), f32),
            jax.ShapeDtypeStruct((B, D), f32),
        ),
        scratch_types=[
            pltpu.VMEM((SPW, 2, HPAD), jnp.int32),
            pltpu.VMEM((SPW * MPAD,), f32),
            pltpu.VMEM((2, L, D), f32),
            pltpu.VMEM((SPW, D), f32),
            pltpu.VMEM((SPW,), f32),
            pltpu.SemaphoreType.DMA,
            pltpu.SemaphoreType.DMA,
        ],
    )
    def sc_kernel(ci, cm, di, dm, wc, wd, oc, od,
                  idx_v, mask_v, rows_v, out_v, denom_v, sem0, sem1):
        wid = lax.axis_index("s") * NC + lax.axis_index("c")
        base = wid * SPW
        sems = (sem0, sem1)

        for idx_hbm, mask_hbm, w_hbm, o_hbm in ((ci, cm, wc, oc), (di, dm, wd, od)):
            pltpu.sync_copy(idx_hbm.at[pl.ds(base, SPW)], idx_v)
            pltpu.sync_copy(mask_hbm.at[pl.ds(base * MPAD, SPW * MPAD)], mask_v)
            _compute_denoms(mask_v, denom_v)

            for cp in _gather_pair(w_hbm, idx_v, rows_v, 0, 0, sem0):
                cp.start()

            def pair_body(t, _):
                k = 2 * t
                for cp in _gather_pair(w_hbm, idx_v, rows_v, k + 1, 1, sem1):
                    cp.start()
                for cp in _gather_pair(w_hbm, idx_v, rows_v, k, 0, sem0):
                    cp.wait()
                _accumulate(rows_v, 0, k, mask_v, denom_v, out_v)

                @pl.when(k + 2 < SPW)
                def _():
                    for cp in _gather_pair(w_hbm, idx_v, rows_v, k + 2, 0, sem0):
                        cp.start()

                for cp in _gather_pair(w_hbm, idx_v, rows_v, k + 1, 1, sem1):
                    cp.wait()
                _accumulate(rows_v, 1, k + 1, mask_v, denom_v, out_v)
                return 0

            lax.fori_loop(0, SPW // 2, pair_body, 0)
            pltpu.sync_copy(out_v, o_hbm.at[pl.ds(base, SPW)])

    return sc_kernel


def kernel(code_vec, code_mask, doc_vec, doc_mask, W_code, W_doc):
    ci = code_vec.astype(jnp.int32).reshape(B, 2, HALF)
    di = doc_vec.astype(jnp.int32).reshape(B, 2, HALF)
    ci = jnp.pad(ci, ((0, 0), (0, 0), (0, HPAD - HALF)))
    di = jnp.pad(di, ((0, 0), (0, 0), (0, HPAD - HALF)))
    cm = jnp.pad(code_mask.astype(jnp.float32), ((0, 0), (0, MPAD - L))).reshape(-1)
    dm = jnp.pad(doc_mask.astype(jnp.float32), ((0, 0), (0, MPAD - L))).reshape(-1)
    enc_code, enc_doc = _make_sc_kernel()(
        ci, cm, di, dm,
        W_code.astype(jnp.float32), W_doc.astype(jnp.float32),
    )
    return (enc_code, enc_doc)
